# baseline (speedup 1.0000x reference)
import functools

import jax
import jax.numpy as jnp
from jax import lax
from jax.experimental import pallas as pl
from jax.experimental.pallas import tpu as pltpu
from jax.experimental.pallas import tpu_sc as plsc

N_CLASSES = 51
D = 576
B = 4096
LAMBDA = 0.03
EPS = 0.09

NC, NS, L = 2, 16, 16
NW = NC * NS
C_CHUNKS = 8
B_CHUNKS = NW // C_CHUNKS
CW = D // C_CHUNKS
BW = B // B_CHUNKS
PHASES = ((0, 8), (8, 32), (40, 32))
N_GROUPS = 4
GV = BW // (N_GROUPS * L)


def _sc_body(xt_hbm, lab_hbm, cfl_hbm, out_hbm, rel_hbm,
             idx_v, xbuf, ctbuf, accv,
             sx0, sx1, sx2, sw0, sw1, sw2, slab, sct):
    wid = lax.axis_index("s") * NC + lax.axis_index("c")
    cc = wid % C_CHUNKS
    bc = wid // C_CHUNKS
    c0 = cc * CW
    b0 = bc * BW

    cp_lab = pltpu.async_copy(lab_hbm.at[pl.ds(b0, BW)], idx_v, slab)
    cp_ct = pltpu.async_copy(cfl_hbm, ctbuf, sct)

    sxs = (sx0, sx1, sx2)
    sws = (sw0, sw1, sw2)
    loads = [
        pltpu.async_copy(
            xt_hbm.at[pl.ds(c0 + ps, pn), pl.ds(b0, BW)],
            xbuf.at[pl.ds(ps, pn)], sxs[p])
        for p, (ps, pn) in enumerate(PHASES)
    ]
    cp_lab.wait()
    cp_ct.wait()

    acc = [jnp.zeros((L,), jnp.float32)] * 4
    wbs = []
    for p, (ps, pn) in enumerate(PHASES):
        loads[p].wait()
        wbs.append(pltpu.async_copy(
            xbuf.at[pl.ds(ps, pn)],
            rel_hbm.at[pl.ds(c0 + ps, pn), pl.ds(b0, BW)], sws[p]))
        for bg in range(N_GROUPS):
            lvds = [idx_v[pl.ds((bg * GV + j) * L, L)] * D for j in range(GV)]

            def c_body(c, carry, _bg=bg, _lvds=lvds):
                cvec = carry[0]
                accs = list(carry[1:])
                for j in range(GV):
                    cv = plsc.load_gather(ctbuf, [_lvds[j] + cvec])
                    xv = xbuf[c, pl.ds((_bg * GV + j) * L, L)]
                    dv = xv - cv
                    accs[j % 4] = accs[j % 4] + dv * dv
                return (cvec + 1, *accs)

            out = lax.fori_loop(
                ps, ps + pn, c_body,
                (jnp.full((L,), c0 + ps, jnp.int32), *acc))
            acc = list(out[1:])

    accv[...] = (acc[0] + acc[1]) + (acc[2] + acc[3])
    pltpu.sync_copy(accv, out_hbm.at[wid])
    for wb in wbs:
        wb.wait()


@functools.cache
def _sc_main():
    return pl.kernel(
        _sc_body,
        out_type=(
            jax.ShapeDtypeStruct((NW, L), jnp.float32),
            jax.ShapeDtypeStruct((D, B), jnp.float32),
        ),
        mesh=plsc.VectorSubcoreMesh(core_axis_name="c", subcore_axis_name="s"),
        scratch_types=[
            pltpu.VMEM((BW,), jnp.int32),
            pltpu.VMEM((CW, BW), jnp.float32),
            pltpu.VMEM((N_CLASSES * D,), jnp.float32),
            pltpu.VMEM((L,), jnp.float32),
            pltpu.SemaphoreType.DMA,
            pltpu.SemaphoreType.DMA,
            pltpu.SemaphoreType.DMA,
            pltpu.SemaphoreType.DMA,
            pltpu.SemaphoreType.DMA,
            pltpu.SemaphoreType.DMA,
            pltpu.SemaphoreType.DMA,
            pltpu.SemaphoreType.DMA,
        ],
        compiler_params=pltpu.CompilerParams(needs_layout_passes=False),
    )


def _tc_pair_body(c_ref, w_ref):
    c = c_ref[...]
    sq = jnp.sum(c * c, axis=1)
    g = lax.dot_general(c, c, (((1,), (1,)), ((), ())),
                        preferred_element_type=jnp.float32,
                        precision=lax.Precision.HIGHEST)
    d2 = sq[:, None] + sq[None, :] - 2.0 * g
    dist = jnp.sqrt(jnp.maximum(d2, 0.0))
    gc = jnp.sum(dist, axis=1) * (1.0 / N_CLASSES)
    mx = jnp.max(gc)
    mn = jnp.min(gc)
    w_ref[...] = (gc - mn + EPS) / (mx - mn)


_tc_pair = pl.pallas_call(
    _tc_pair_body,
    out_shape=jax.ShapeDtypeStruct((N_CLASSES,), jnp.float32),
)


def _tc_loss_body(p_ref, l_ref):
    loss = jnp.sum(p_ref[...]) * (LAMBDA / (B * D))
    l_ref[...] = jnp.reshape(loss, (1, 1))


_tc_loss = pl.pallas_call(
    _tc_loss_body,
    out_shape=jax.ShapeDtypeStruct((1, 1), jnp.float32),
)


def kernel(relation_logits_raw, rel_labels, centers):
    labels = rel_labels.astype(jnp.int32)
    xt = jnp.swapaxes(relation_logits_raw, 0, 1)
    cflat = jnp.reshape(centers, (-1,))
    parts, rel_t = _sc_main()(xt, labels, cflat)
    weight = _tc_pair(centers)
    loss = _tc_loss(parts)
    return (weight, loss[0, 0], jnp.swapaxes(rel_t, 0, 1))


# speedup vs baseline: 2.2821x; 2.2821x over previous
import functools

import jax
import jax.numpy as jnp
from jax import lax
from jax.experimental import pallas as pl
from jax.experimental.pallas import tpu as pltpu
from jax.experimental.pallas import tpu_sc as plsc

N_CLASSES = 51
D = 576
B = 4096
LAMBDA = 0.03
EPS = 0.09

NC, NS, L = 2, 16, 16
NW = NC * NS
C_CHUNKS = 8
B_CHUNKS = NW // C_CHUNKS
CW = D // C_CHUNKS
BW = B // B_CHUNKS
PHASES = ((0, 8), (8, 32), (40, 32))
N_GROUPS = 4
GV = BW // (N_GROUPS * L)


def _sc_body(xt_hbm, lab_hbm, cfl_hbm, out_hbm, rel_hbm,
             idx_v, xbuf, ctbuf, accv,
             sx0, sx1, sx2, sw0, sw1, sw2, slab, sct):
    wid = lax.axis_index("s") * NC + lax.axis_index("c")
    cc = wid % C_CHUNKS
    bc = wid // C_CHUNKS
    c0 = cc * CW
    b0 = bc * BW

    cp_lab = pltpu.async_copy(lab_hbm.at[pl.ds(b0, BW)], idx_v, slab)
    cp_ct = pltpu.async_copy(
        cfl_hbm.at[pl.ds(c0 * N_CLASSES, CW * N_CLASSES)], ctbuf, sct)

    sxs = (sx0, sx1, sx2)
    sws = (sw0, sw1, sw2)
    loads = [
        pltpu.async_copy(
            xt_hbm.at[pl.ds(c0 + ps, pn), pl.ds(b0, BW)],
            xbuf.at[pl.ds(ps, pn)], sxs[p])
        for p, (ps, pn) in enumerate(PHASES)
    ]
    cp_lab.wait()
    cp_ct.wait()

    acc = [jnp.zeros((L,), jnp.float32)] * 4
    wbs = []
    for p, (ps, pn) in enumerate(PHASES):
        loads[p].wait()
        wbs.append(pltpu.async_copy(
            xbuf.at[pl.ds(ps, pn)],
            rel_hbm.at[pl.ds(c0 + ps, pn), pl.ds(b0, BW)], sws[p]))
        for bg in range(N_GROUPS):
            lvs = [idx_v[pl.ds((bg * GV + j) * L, L)] for j in range(GV)]

            def c_body(c, carry, _bg=bg, _lvs=lvs):
                cvec = carry[0]
                accs = list(carry[1:])
                for j in range(GV):
                    cv = plsc.load_gather(ctbuf, [cvec + _lvs[j]])
                    xv = xbuf[c, pl.ds((_bg * GV + j) * L, L)]
                    dv = xv - cv
                    accs[j % 4] = accs[j % 4] + dv * dv
                return (cvec + N_CLASSES, *accs)

            out = lax.fori_loop(
                ps, ps + pn, c_body,
                (jnp.full((L,), ps * N_CLASSES, jnp.int32), *acc))
            acc = list(out[1:])

    accv[...] = (acc[0] + acc[1]) + (acc[2] + acc[3])
    pltpu.sync_copy(accv, out_hbm.at[wid])
    for wb in wbs:
        wb.wait()


@functools.cache
def _sc_main():
    return pl.kernel(
        _sc_body,
        out_type=(
            jax.ShapeDtypeStruct((NW, L), jnp.float32),
            jax.ShapeDtypeStruct((D, B), jnp.float32),
        ),
        mesh=plsc.VectorSubcoreMesh(core_axis_name="c", subcore_axis_name="s"),
        scratch_types=[
            pltpu.VMEM((BW,), jnp.int32),
            pltpu.VMEM((CW, BW), jnp.float32),
            pltpu.VMEM((CW * N_CLASSES,), jnp.float32),
            pltpu.VMEM((L,), jnp.float32),
            pltpu.SemaphoreType.DMA,
            pltpu.SemaphoreType.DMA,
            pltpu.SemaphoreType.DMA,
            pltpu.SemaphoreType.DMA,
            pltpu.SemaphoreType.DMA,
            pltpu.SemaphoreType.DMA,
            pltpu.SemaphoreType.DMA,
            pltpu.SemaphoreType.DMA,
        ],
        compiler_params=pltpu.CompilerParams(needs_layout_passes=False),
    )


def _tc_pair_body(c_ref, w_ref):
    c = c_ref[...]
    sq = jnp.sum(c * c, axis=1)
    g = lax.dot_general(c, c, (((1,), (1,)), ((), ())),
                        preferred_element_type=jnp.float32,
                        precision=lax.Precision.HIGHEST)
    d2 = sq[:, None] + sq[None, :] - 2.0 * g
    dist = jnp.sqrt(jnp.maximum(d2, 0.0))
    gc = jnp.sum(dist, axis=1) * (1.0 / N_CLASSES)
    mx = jnp.max(gc)
    mn = jnp.min(gc)
    w_ref[...] = (gc - mn + EPS) / (mx - mn)


_tc_pair = pl.pallas_call(
    _tc_pair_body,
    out_shape=jax.ShapeDtypeStruct((N_CLASSES,), jnp.float32),
)


def _tc_loss_body(p_ref, l_ref):
    loss = jnp.sum(p_ref[...]) * (LAMBDA / (B * D))
    l_ref[...] = jnp.reshape(loss, (1, 1))


_tc_loss = pl.pallas_call(
    _tc_loss_body,
    out_shape=jax.ShapeDtypeStruct((1, 1), jnp.float32),
)


def kernel(relation_logits_raw, rel_labels, centers):
    labels = rel_labels.astype(jnp.int32)
    xt = jnp.swapaxes(relation_logits_raw, 0, 1)
    cflat = lax.reshape(centers, (N_CLASSES * D,), dimensions=(1, 0))
    parts, rel_t = _sc_main()(xt, labels, cflat)
    weight = _tc_pair(centers)
    loss = _tc_loss(parts)
    return (weight, loss[0, 0], jnp.swapaxes(rel_t, 0, 1))
